# Initial kernel scaffold; baseline (speedup 1.0000x reference)
import jax
import jax.numpy as jnp
from jax.experimental import pallas as pl


def kernel(x, adj, W_dense, b_dense, W_gat, att_src, att_dst, b_gat):
    raise NotImplementedError("write your pallas kernel here")


# speedup vs baseline: 17.9837x; 17.9837x over previous
import functools

import jax
import jax.numpy as jnp
from jax import lax
from jax.experimental import pallas as pl
from jax.experimental.pallas import tpu as pltpu
from jax.experimental.pallas import tpu_sc as plsc

N = 10000
NPAD = 10240
IN_DIM = 128
EMB = 64
E_RAW = 320000
E_TOT = E_RAW + N
E_PAD = 330240
CHUNK = 688
GRP = CHUNK // 16
P1_PER_TILE = E_PAD // 16
P1_CHUNKS = P1_PER_TILE // CHUNK
P2_PER_TILE = E_PAD // 32
P2_CHUNKS = P2_PER_TILE // CHUNK
SLICE = NPAD // 16
BR = 1024


def _tc1_body(x_ref, wd_ref, bd_ref, wg_ref, asv_ref, adv_ref,
              hp_ref, as_ref, ad_ref, c_ref):
    h = lax.dot_general(x_ref[...], wd_ref[...], (((1,), (1,)), ((), ())),
                        preferred_element_type=jnp.float32)
    h = jnp.maximum(h + bd_ref[...], 0.0)
    hp = lax.dot_general(h, wg_ref[...], (((1,), (1,)), ((), ())),
                         preferred_element_type=jnp.float32)
    hp_ref[...] = hp
    a_s = jnp.sum(hp * asv_ref[...], axis=1)
    a_d = jnp.sum(hp * adv_ref[...], axis=1)
    as_ref[...] = a_s[None, :]
    ad_ref[...] = a_d[None, :]
    m = jnp.max(a_s) + jnp.max(a_d)
    c = jnp.where(m >= 0.0, m, 0.2 * m)
    c_ref[...] = jnp.full((1, 16), c, jnp.float32)


def _sc_body(src_h, dst_h, as_h, ad_h, hp_h, c_h, z1_h, z2_h, out_h,
             asrc_v, adst_v, denom_v, rows_v, sidx_v, didx_v, pbuf_v, cv_v,
             denom_s, embed_s, sem):
    cid = lax.axis_index("c")
    sid = lax.axis_index("s")
    wid = cid * 16 + sid

    pltpu.sync_copy(as_h.at[0], asrc_v.at[pl.ds(0, N)])
    pltpu.sync_copy(ad_h.at[0], adst_v.at[pl.ds(0, N)])
    pltpu.sync_copy(c_h.at[0], cv_v)
    adst_v[pl.ds(N, 16)] = jnp.zeros((16,), jnp.float32)
    pltpu.sync_copy(z1_h, denom_s.at[pl.ds(sid * SLICE, SLICE)])
    pltpu.sync_copy(z2_h, embed_s.at[pl.ds(sid * SLICE, SLICE)])
    plsc.subcore_barrier()

    cvec = cv_v[...]

    def p1_chunk(i, carry):
        off = sid * P1_PER_TILE + i * CHUNK
        pltpu.sync_copy(src_h.at[pl.ds(off, CHUNK)], sidx_v)
        pltpu.sync_copy(dst_h.at[pl.ds(off, CHUNK)], didx_v)

        def grp(g, c2):
            si = sidx_v[pl.ds(g * 16, 16)]
            di = didx_v[pl.ds(g * 16, 16)]
            e = plsc.load_gather(asrc_v, [si]) + plsc.load_gather(adst_v, [di])
            e = jnp.where(e >= 0.0, e, 0.2 * e)
            pbuf_v[pl.ds(g * 16, 16)] = jnp.exp(e - cvec)
            return c2

        lax.fori_loop(0, GRP, grp, 0)
        pltpu.sync_copy(pbuf_v, denom_s.at[didx_v], add=True)
        return carry

    lax.fori_loop(0, P1_CHUNKS, p1_chunk, 0)
    plsc.subcore_barrier()
    pltpu.sync_copy(denom_s, denom_v)

    iotas = [lax.iota(jnp.int32, 16) + (16 * c) for c in range(4)]

    def p2_chunk(i, carry):
        off = wid * P2_PER_TILE + i * CHUNK
        pltpu.sync_copy(src_h.at[pl.ds(off, CHUNK)], sidx_v)
        pltpu.sync_copy(dst_h.at[pl.ds(off, CHUNK)], didx_v)
        pltpu.async_copy(hp_h.at[sidx_v], rows_v, sem).wait()

        def grp(g, c2):
            si = sidx_v[pl.ds(g * 16, 16)]
            di = didx_v[pl.ds(g * 16, 16)]
            e = plsc.load_gather(asrc_v, [si]) + plsc.load_gather(adst_v, [di])
            e = jnp.where(e >= 0.0, e, 0.2 * e)
            p = jnp.exp(e - cvec)
            dn = plsc.load_gather(denom_v, [di])
            pbuf_v[pl.ds(g * 16, 16)] = p / (dn + 1e-16)
            return c2

        lax.fori_loop(0, GRP, grp, 0)

        def scale(e_i, c2):
            ridx = jnp.zeros((16,), jnp.int32) + e_i
            al = plsc.load_gather(pbuf_v, [ridx])
            for cb in range(4):
                v = plsc.load_gather(rows_v, [ridx, iotas[cb]])
                plsc.store_scatter(rows_v, [ridx, iotas[cb]], v * al)
            return c2

        lax.fori_loop(0, CHUNK, scale, 0)
        pltpu.sync_copy(rows_v, embed_s.at[didx_v], add=True)
        return carry

    lax.fori_loop(0, P2_CHUNKS, p2_chunk, 0)
    plsc.subcore_barrier()
    pltpu.sync_copy(embed_s.at[pl.ds(sid * SLICE, SLICE)],
                    out_h.at[cid, pl.ds(sid * SLICE, SLICE)])


def _tc2_body(pr0, pr1, pc0, pc1, bg, out_ref, emb_ref):
    er = pr0[...] + pr1[...] + bg[...]
    ec = pc0[...] + pc1[...] + bg[...]
    logits = lax.dot_general(er, ec, (((1,), (1,)), ((), ())),
                             preferred_element_type=jnp.float32)
    out_ref[...] = jax.nn.sigmoid(logits)

    @pl.when(pl.program_id(1) == 0)
    def _():
        emb_ref[...] = er


def kernel(x, adj, W_dense, b_dense, W_gat, att_src, att_dst, b_gat):
    f32 = jnp.float32
    loop = jnp.arange(N, dtype=adj.dtype)
    pad = E_PAD - E_TOT
    src = jnp.concatenate([adj[0].astype(jnp.int32), loop.astype(jnp.int32),
                           jnp.zeros((pad,), jnp.int32)])
    dst = jnp.concatenate([adj[1].astype(jnp.int32), loop.astype(jnp.int32),
                           jnp.full((pad,), N, jnp.int32)])

    hp, a_s, a_d, c_c = pl.pallas_call(
        _tc1_body,
        out_shape=[
            jax.ShapeDtypeStruct((N, EMB), f32),
            jax.ShapeDtypeStruct((1, N), f32),
            jax.ShapeDtypeStruct((1, N), f32),
            jax.ShapeDtypeStruct((1, 16), f32),
        ],
    )(x, W_dense, b_dense.reshape(1, EMB), W_gat,
      att_src.reshape(1, EMB), att_dst.reshape(1, EMB))

    mesh = plsc.VectorSubcoreMesh(core_axis_name="c", subcore_axis_name="s",
                                  num_cores=2, num_subcores=16)
    sc_fn = pl.kernel(
        _sc_body,
        out_type=jax.ShapeDtypeStruct((2, NPAD, EMB), f32),
        mesh=mesh,
        compiler_params=pltpu.CompilerParams(needs_layout_passes=False,
                                             use_tc_tiling_on_sc=False),
        scratch_types=[
            pltpu.VMEM((NPAD,), f32),
            pltpu.VMEM((NPAD,), f32),
            pltpu.VMEM((NPAD,), f32),
            pltpu.VMEM((CHUNK, EMB), f32),
            pltpu.VMEM((CHUNK,), jnp.int32),
            pltpu.VMEM((CHUNK,), jnp.int32),
            pltpu.VMEM((CHUNK,), f32),
            pltpu.VMEM((16,), f32),
            pltpu.VMEM_SHARED((NPAD,), f32),
            pltpu.VMEM_SHARED((NPAD, EMB), f32),
            pltpu.SemaphoreType.DMA,
        ],
    )
    z1 = jnp.zeros((SLICE,), f32)
    z2 = jnp.zeros((SLICE, EMB), f32)
    partials = sc_fn(src, dst, a_s, a_d, hp, c_c, z1, z2)

    out, embed = pl.pallas_call(
        _tc2_body,
        grid=(NPAD // BR, NPAD // BR),
        in_specs=[
            pl.BlockSpec((BR, EMB), lambda i, j: (i, 0)),
            pl.BlockSpec((BR, EMB), lambda i, j: (i, 0)),
            pl.BlockSpec((BR, EMB), lambda i, j: (j, 0)),
            pl.BlockSpec((BR, EMB), lambda i, j: (j, 0)),
            pl.BlockSpec((1, EMB), lambda i, j: (0, 0)),
        ],
        out_specs=[
            pl.BlockSpec((BR, BR), lambda i, j: (i, j)),
            pl.BlockSpec((BR, EMB), lambda i, j: (i, 0)),
        ],
        out_shape=[
            jax.ShapeDtypeStruct((N, N), f32),
            jax.ShapeDtypeStruct((N, EMB), f32),
        ],
    )(partials[0], partials[1], partials[0], partials[1], b_gat.reshape(1, EMB))
    return (out, embed)


# speedup vs baseline: 19.4366x; 1.0808x over previous
import functools

import jax
import jax.numpy as jnp
from jax import lax
from jax.experimental import pallas as pl
from jax.experimental.pallas import tpu as pltpu
from jax.experimental.pallas import tpu_sc as plsc

N = 10000
NPAD = 10240
IN_DIM = 128
EMB = 64
E_RAW = 320000
E_TOT = E_RAW + N
CHUNK = 352
GRP = CHUNK // 16
P2_CHUNKS = 30
P2_PER_TILE = P2_CHUNKS * CHUNK
E_PAD = 32 * P2_PER_TILE
P1_PER_TILE = E_PAD // 16
P1_CHUNKS = P1_PER_TILE // CHUNK
SLICE = NPAD // 16
AB = 2 * N + 32
BR = 1024


def _tc1_body(x_ref, wd_ref, bd_ref, wg_ref, att2_ref, hp_ref, ab_ref, c_ref):
    h = lax.dot_general(x_ref[...], wd_ref[...], (((1,), (1,)), ((), ())),
                        preferred_element_type=jnp.float32)
    h = jnp.maximum(h + bd_ref[...], 0.0)
    hp = lax.dot_general(h, wg_ref[...], (((1,), (1,)), ((), ())),
                         preferred_element_type=jnp.float32)
    hp_ref[...] = hp
    ab = lax.dot_general(hp, att2_ref[...], (((1,), (1,)), ((), ())),
                         preferred_element_type=jnp.float32)
    ab_ref[...] = ab
    m = jnp.max(ab, axis=0)
    c = m[0] + m[1]
    c = jnp.where(c >= 0.0, c, 0.2 * c)
    c_ref[...] = jnp.full((1, 16), c, jnp.float32)


def _sc_body(src_h, dst_h, ab_h, hp_h, c_h, z1_h, z2_h, out_h,
             ab_v, denom_v, rows0_v, rows1_v, sidx0_v, sidx1_v,
             didx0_v, didx1_v, pbuf_v, cv_v, denom_s, embed_s,
             gsem0, gsem1, isem0, isem1):
    cid = lax.axis_index("c")
    sid = lax.axis_index("s")
    wid = cid * 16 + sid

    pltpu.sync_copy(ab_h, ab_v.at[pl.ds(0, AB)])
    pltpu.sync_copy(c_h.at[0], cv_v)
    pltpu.sync_copy(z1_h, denom_s.at[pl.ds(sid * SLICE, SLICE)])
    pltpu.sync_copy(z2_h, embed_s.at[pl.ds(sid * SLICE, SLICE)])
    plsc.subcore_barrier()

    cvec = cv_v[...]
    bufs = ((sidx0_v, didx0_v, rows0_v, gsem0, isem0),
            (sidx1_v, didx1_v, rows1_v, gsem1, isem1))
    iotas = [lax.iota(jnp.int32, 16) + (16 * c) for c in range(4)]

    def edge_p(sidx, didx, g):
        si = sidx[pl.ds(g * 16, 16)]
        di = didx[pl.ds(g * 16, 16)]
        e = (plsc.load_gather(ab_v, [si + si])
             + plsc.load_gather(ab_v, [di + di + 1]))
        e = jnp.where(e >= 0.0, e, 0.2 * e)
        return di, jnp.exp(e - cvec)

    p1_base = sid * P1_PER_TILE
    for b in range(2):
        sidx, didx, _, _, isem = bufs[b]
        off = p1_base + b * CHUNK
        pltpu.async_copy(src_h.at[pl.ds(off, CHUNK)], sidx, isem)
        pltpu.async_copy(dst_h.at[pl.ds(off, CHUNK)], didx, isem)

    def p1_iter(o, carry):
        for b in range(2):
            i = o * 2 + b
            sidx, didx, _, _, isem = bufs[b]
            pltpu.make_async_copy(src_h.at[pl.ds(0, CHUNK)], sidx, isem).wait()
            pltpu.make_async_copy(dst_h.at[pl.ds(0, CHUNK)], didx, isem).wait()

            def grp(g, c2):
                _, p = edge_p(sidx, didx, g)
                pbuf_v[pl.ds(g * 16, 16)] = p
                return c2

            lax.fori_loop(0, GRP, grp, 0)
            pltpu.sync_copy(pbuf_v, denom_s.at[didx], add=True)

            @pl.when(i < P1_CHUNKS - 2)
            def _():
                off = p1_base + (i + 2) * CHUNK
                pltpu.async_copy(src_h.at[pl.ds(off, CHUNK)], sidx, isem)
                pltpu.async_copy(dst_h.at[pl.ds(off, CHUNK)], didx, isem)
        return carry

    lax.fori_loop(0, P1_CHUNKS // 2, p1_iter, 0)
    plsc.subcore_barrier()
    pltpu.sync_copy(denom_s, denom_v)

    p2_base = wid * P2_PER_TILE

    def p2_fetch(i, b):
        sidx, didx, rows, gsem, _ = bufs[b]
        off = p2_base + i * CHUNK
        pltpu.sync_copy(src_h.at[pl.ds(off, CHUNK)], sidx)
        pltpu.sync_copy(dst_h.at[pl.ds(off, CHUNK)], didx)
        pltpu.async_copy(hp_h.at[sidx], rows, gsem)

    def p2_chunk(i, b):
        sidx, didx, rows, gsem, _ = bufs[b]
        pltpu.make_async_copy(hp_h.at[sidx], rows, gsem).wait()

        def grp(g, c2):
            di, p = edge_p(sidx, didx, g)
            alpha = p / (plsc.load_gather(denom_v, [di]) + 1e-16)
            rbase = jnp.zeros((16,), jnp.int32) + g * 16
            for j in range(16):
                aj = alpha.at[jnp.full((16,), j, jnp.int32)].get(
                    mode="promise_in_bounds")
                ridx = rbase + j
                for cb in range(4):
                    v = plsc.load_gather(rows, [ridx, iotas[cb]])
                    plsc.store_scatter(rows, [ridx, iotas[cb]], v * aj)
            return c2

        lax.fori_loop(0, GRP, grp, 0)
        pltpu.sync_copy(rows, embed_s.at[didx], add=True)

        @pl.when(i < P2_CHUNKS - 2)
        def _():
            p2_fetch(i + 2, b)

    p2_fetch(0, 0)
    p2_fetch(1, 1)

    def p2_iter(o, carry):
        p2_chunk(o * 2, 0)
        p2_chunk(o * 2 + 1, 1)
        return carry

    lax.fori_loop(0, P2_CHUNKS // 2, p2_iter, 0)

    plsc.subcore_barrier()
    pltpu.sync_copy(embed_s.at[pl.ds(sid * SLICE, SLICE)],
                    out_h.at[cid, pl.ds(sid * SLICE, SLICE)])


def _tc2_body(pr0, pr1, pc0, pc1, bg, out_ref, emb_ref):
    er = pr0[...] + pr1[...] + bg[...]
    ec = pc0[...] + pc1[...] + bg[...]
    logits = lax.dot_general(er, ec, (((1,), (1,)), ((), ())),
                             preferred_element_type=jnp.float32)
    out_ref[...] = jax.nn.sigmoid(logits)

    @pl.when(pl.program_id(1) == 0)
    def _():
        emb_ref[...] = er


def kernel(x, adj, W_dense, b_dense, W_gat, att_src, att_dst, b_gat):
    f32 = jnp.float32
    loop = jnp.arange(N, dtype=adj.dtype)
    pad = E_PAD - E_TOT
    src = jnp.concatenate([adj[0].astype(jnp.int32), loop.astype(jnp.int32),
                           jnp.zeros((pad,), jnp.int32)])
    dst = jnp.concatenate([adj[1].astype(jnp.int32), loop.astype(jnp.int32),
                           jnp.full((pad,), N, jnp.int32)])

    hp, ab, c_c = pl.pallas_call(
        _tc1_body,
        out_shape=[
            jax.ShapeDtypeStruct((N, EMB), f32),
            jax.ShapeDtypeStruct((N, 2), f32),
            jax.ShapeDtypeStruct((1, 16), f32),
        ],
    )(x, W_dense, b_dense.reshape(1, EMB), W_gat,
      jnp.stack([att_src, att_dst], axis=0))
    ab_flat = jnp.concatenate([ab, jnp.zeros((16, 2), f32)]).reshape(-1)

    mesh = plsc.VectorSubcoreMesh(core_axis_name="c", subcore_axis_name="s",
                                  num_cores=2, num_subcores=16)
    sc_fn = pl.kernel(
        _sc_body,
        out_type=jax.ShapeDtypeStruct((2, NPAD, EMB), f32),
        mesh=mesh,
        compiler_params=pltpu.CompilerParams(needs_layout_passes=False,
                                             use_tc_tiling_on_sc=False),
        scratch_types=[
            pltpu.VMEM((AB,), f32),
            pltpu.VMEM((NPAD,), f32),
            pltpu.VMEM((CHUNK, EMB), f32),
            pltpu.VMEM((CHUNK, EMB), f32),
            pltpu.VMEM((CHUNK,), jnp.int32),
            pltpu.VMEM((CHUNK,), jnp.int32),
            pltpu.VMEM((CHUNK,), jnp.int32),
            pltpu.VMEM((CHUNK,), jnp.int32),
            pltpu.VMEM((CHUNK,), f32),
            pltpu.VMEM((16,), f32),
            pltpu.VMEM_SHARED((NPAD,), f32),
            pltpu.VMEM_SHARED((NPAD, EMB), f32),
            pltpu.SemaphoreType.DMA,
            pltpu.SemaphoreType.DMA,
            pltpu.SemaphoreType.DMA,
            pltpu.SemaphoreType.DMA,
        ],
    )
    z1 = jnp.zeros((SLICE,), f32)
    z2 = jnp.zeros((SLICE, EMB), f32)
    partials = sc_fn(src, dst, ab_flat, hp, c_c, z1, z2)

    out, embed = pl.pallas_call(
        _tc2_body,
        grid=(NPAD // BR, NPAD // BR),
        in_specs=[
            pl.BlockSpec((BR, EMB), lambda i, j: (i, 0)),
            pl.BlockSpec((BR, EMB), lambda i, j: (i, 0)),
            pl.BlockSpec((BR, EMB), lambda i, j: (j, 0)),
            pl.BlockSpec((BR, EMB), lambda i, j: (j, 0)),
            pl.BlockSpec((1, EMB), lambda i, j: (0, 0)),
        ],
        out_specs=[
            pl.BlockSpec((BR, BR), lambda i, j: (i, j)),
            pl.BlockSpec((BR, EMB), lambda i, j: (i, 0)),
        ],
        out_shape=[
            jax.ShapeDtypeStruct((N, N), f32),
            jax.ShapeDtypeStruct((N, EMB), f32),
        ],
    )(partials[0], partials[1], partials[0], partials[1], b_gat.reshape(1, EMB))
    return (out, embed)


# speedup vs baseline: 20.3387x; 1.0464x over previous
import functools

import jax
import jax.numpy as jnp
from jax import lax
from jax.experimental import pallas as pl
from jax.experimental.pallas import tpu as pltpu
from jax.experimental.pallas import tpu_sc as plsc

N = 10000
NPAD = 10240
IN_DIM = 128
EMB = 64
E_RAW = 320000
E_TOT = E_RAW + N
CHUNK = 320
GRP = CHUNK // 16
P2_CHUNKS = 33
P2_PER_TILE = P2_CHUNKS * CHUNK
E_PAD = 32 * P2_PER_TILE
SLICE = NPAD // 16
AB = 2 * N + 32
BR = 1024


def _tc1_body(x_ref, wd_ref, bd_ref, wg_ref, att2_ref, hp_ref, ab_ref, c_ref):
    h = lax.dot_general(x_ref[...], wd_ref[...], (((1,), (1,)), ((), ())),
                        preferred_element_type=jnp.float32)
    h = jnp.maximum(h + bd_ref[...], 0.0)
    hp = lax.dot_general(h, wg_ref[...], (((1,), (1,)), ((), ())),
                         preferred_element_type=jnp.float32)
    hp_ref[...] = hp
    ab = lax.dot_general(hp, att2_ref[...], (((1,), (1,)), ((), ())),
                         preferred_element_type=jnp.float32)
    ab_ref[...] = ab
    m = jnp.max(ab, axis=0)
    c = m[0] + m[1]
    c = jnp.where(c >= 0.0, c, 0.2 * c)
    c_ref[...] = jnp.full((1, 16), c, jnp.float32)


def _sc_body(src_h, dst_h, ab_h, hp_h, c_h, z1_h, z2_h, out_e, out_d,
             ab_v, r0_v, r1_v, r2_v, s0_v, s1_v, s2_v, d0_v, d1_v, d2_v,
             p0_v, p1_v, p2_v, cv_v, denom_s, embed_s,
             g0, g1, g2, x0, x1, x2):
    cid = lax.axis_index("c")
    sid = lax.axis_index("s")
    wid = cid * 16 + sid
    base = wid * P2_PER_TILE

    pltpu.sync_copy(ab_h, ab_v)
    pltpu.sync_copy(c_h.at[0], cv_v)
    pltpu.sync_copy(z1_h, denom_s.at[pl.ds(sid * SLICE, SLICE)])
    pltpu.sync_copy(z2_h, embed_s.at[pl.ds(sid * SLICE, SLICE)])
    plsc.subcore_barrier()

    cvec = cv_v[...]
    slots = ((s0_v, d0_v, r0_v, p0_v, g0, x0),
             (s1_v, d1_v, r1_v, p1_v, g1, x1),
             (s2_v, d2_v, r2_v, p2_v, g2, x2))
    iotas = [lax.iota(jnp.int32, 16) + (16 * c) for c in range(4)]

    def drain_scatters(k):
        sidx, didx, rows, pb, gsem, ssem = slots[k]
        pltpu.make_async_copy(rows, embed_s.at[didx], ssem).wait()
        pltpu.make_async_copy(pb, denom_s.at[didx], ssem).wait()

    def fetch(i, k):
        sidx, didx, rows, pb, gsem, ssem = slots[k]
        off = base + i * CHUNK
        pltpu.sync_copy(src_h.at[pl.ds(off, CHUNK)], sidx)
        pltpu.sync_copy(dst_h.at[pl.ds(off, CHUNK)], didx)
        pltpu.async_copy(hp_h.at[sidx], rows, gsem)

    def process(i, k):
        sidx, didx, rows, pb, gsem, ssem = slots[k]
        pltpu.make_async_copy(hp_h.at[sidx], rows, gsem).wait()

        def grp(g, c2):
            si = sidx[pl.ds(g * 16, 16)]
            di = didx[pl.ds(g * 16, 16)]
            e = (plsc.load_gather(ab_v, [si + si])
                 + plsc.load_gather(ab_v, [di + di + 1]))
            e = jnp.where(e >= 0.0, e, 0.2 * e)
            p = jnp.exp(e - cvec)
            pb[pl.ds(g * 16, 16)] = p
            rbase = jnp.zeros((16,), jnp.int32) + g * 16
            for j in range(16):
                aj = p.at[jnp.full((16,), j, jnp.int32)].get(
                    mode="promise_in_bounds")
                ridx = rbase + j
                for cb in range(4):
                    v = plsc.load_gather(rows, [ridx, iotas[cb]])
                    plsc.store_scatter(rows, [ridx, iotas[cb]], v * aj)
            return c2

        lax.fori_loop(0, GRP, grp, 0)
        pltpu.async_copy(rows, embed_s.at[didx], ssem, add=True)
        pltpu.async_copy(pb, denom_s.at[didx], ssem, add=True)

    fetch(0, 0)
    fetch(1, 1)

    def ring_iter(o, carry):
        for k in range(3):
            i = o * 3 + k
            process(i, k)
            nk = (k + 2) % 3

            @pl.when(i < P2_CHUNKS - 2)
            def _():
                @pl.when(i >= 1)
                def _():
                    drain_scatters(nk)
                fetch(i + 2, nk)
        return carry

    lax.fori_loop(0, P2_CHUNKS // 3, ring_iter, 0)
    for k in range(3):
        drain_scatters(k)

    plsc.subcore_barrier()
    pltpu.sync_copy(embed_s.at[pl.ds(sid * SLICE, SLICE)],
                    out_e.at[cid, pl.ds(sid * SLICE, SLICE)])
    pltpu.sync_copy(denom_s.at[pl.ds(sid * SLICE, SLICE)],
                    out_d.at[cid, pl.ds(sid * SLICE, SLICE)])


def _tc2_body(pr0, pr1, pc0, pc1, dn0, dn1, bg, out_ref, emb_ref):
    i = pl.program_id(0)
    j = pl.program_id(1)
    dr = dn0[pl.ds(i * BR, BR), :] + dn1[pl.ds(i * BR, BR), :] + 1e-16
    dc = dn0[pl.ds(j * BR, BR), :] + dn1[pl.ds(j * BR, BR), :] + 1e-16
    er = (pr0[...] + pr1[...]) / dr + bg[...]
    ec = (pc0[...] + pc1[...]) / dc + bg[...]
    logits = lax.dot_general(er, ec, (((1,), (1,)), ((), ())),
                             preferred_element_type=jnp.float32)
    out_ref[...] = jax.nn.sigmoid(logits)

    @pl.when(j == 0)
    def _():
        emb_ref[...] = er


def kernel(x, adj, W_dense, b_dense, W_gat, att_src, att_dst, b_gat):
    f32 = jnp.float32
    loop = jnp.arange(N, dtype=adj.dtype)
    pad = E_PAD - E_TOT
    src = jnp.concatenate([adj[0].astype(jnp.int32), loop.astype(jnp.int32),
                           jnp.zeros((pad,), jnp.int32)])
    dst = jnp.concatenate([adj[1].astype(jnp.int32), loop.astype(jnp.int32),
                           jnp.full((pad,), N, jnp.int32)])

    hp, ab, c_c = pl.pallas_call(
        _tc1_body,
        out_shape=[
            jax.ShapeDtypeStruct((N, EMB), f32),
            jax.ShapeDtypeStruct((N, 2), f32),
            jax.ShapeDtypeStruct((1, 16), f32),
        ],
    )(x, W_dense, b_dense.reshape(1, EMB), W_gat,
      jnp.stack([att_src, att_dst], axis=0))
    ab_flat = jnp.concatenate([ab, jnp.zeros((16, 2), f32)]).reshape(-1)

    mesh = plsc.VectorSubcoreMesh(core_axis_name="c", subcore_axis_name="s",
                                  num_cores=2, num_subcores=16)
    sc_fn = pl.kernel(
        _sc_body,
        out_type=[
            jax.ShapeDtypeStruct((2, NPAD, EMB), f32),
            jax.ShapeDtypeStruct((2, NPAD), f32),
        ],
        mesh=mesh,
        compiler_params=pltpu.CompilerParams(needs_layout_passes=False,
                                             use_tc_tiling_on_sc=False),
        scratch_types=[
            pltpu.VMEM((AB,), f32),
            pltpu.VMEM((CHUNK, EMB), f32),
            pltpu.VMEM((CHUNK, EMB), f32),
            pltpu.VMEM((CHUNK, EMB), f32),
            pltpu.VMEM((CHUNK,), jnp.int32),
            pltpu.VMEM((CHUNK,), jnp.int32),
            pltpu.VMEM((CHUNK,), jnp.int32),
            pltpu.VMEM((CHUNK,), jnp.int32),
            pltpu.VMEM((CHUNK,), jnp.int32),
            pltpu.VMEM((CHUNK,), jnp.int32),
            pltpu.VMEM((CHUNK,), f32),
            pltpu.VMEM((CHUNK,), f32),
            pltpu.VMEM((CHUNK,), f32),
            pltpu.VMEM((16,), f32),
            pltpu.VMEM_SHARED((NPAD,), f32),
            pltpu.VMEM_SHARED((NPAD, EMB), f32),
            pltpu.SemaphoreType.DMA,
            pltpu.SemaphoreType.DMA,
            pltpu.SemaphoreType.DMA,
            pltpu.SemaphoreType.DMA,
            pltpu.SemaphoreType.DMA,
            pltpu.SemaphoreType.DMA,
        ],
    )
    z1 = jnp.zeros((SLICE,), f32)
    z2 = jnp.zeros((SLICE, EMB), f32)
    nums, dens = sc_fn(src, dst, ab_flat, hp, c_c, z1, z2)

    out, embed = pl.pallas_call(
        _tc2_body,
        grid=(NPAD // BR, NPAD // BR),
        in_specs=[
            pl.BlockSpec((BR, EMB), lambda i, j: (i, 0)),
            pl.BlockSpec((BR, EMB), lambda i, j: (i, 0)),
            pl.BlockSpec((BR, EMB), lambda i, j: (j, 0)),
            pl.BlockSpec((BR, EMB), lambda i, j: (j, 0)),
            pl.BlockSpec((NPAD, 1), lambda i, j: (0, 0)),
            pl.BlockSpec((NPAD, 1), lambda i, j: (0, 0)),
            pl.BlockSpec((1, EMB), lambda i, j: (0, 0)),
        ],
        out_specs=[
            pl.BlockSpec((BR, BR), lambda i, j: (i, j)),
            pl.BlockSpec((BR, EMB), lambda i, j: (i, 0)),
        ],
        out_shape=[
            jax.ShapeDtypeStruct((N, N), f32),
            jax.ShapeDtypeStruct((N, EMB), f32),
        ],
    )(nums[0], nums[1], nums[0], nums[1],
      dens[0].reshape(NPAD, 1), dens[1].reshape(NPAD, 1), b_gat.reshape(1, EMB))
    return (out, embed)


# speedup vs baseline: 22.1310x; 1.0881x over previous
import functools

import jax
import jax.numpy as jnp
from jax import lax
from jax.experimental import pallas as pl
from jax.experimental.pallas import tpu as pltpu
from jax.experimental.pallas import tpu_sc as plsc

N = 10000
NPAD = 10240
IN_DIM = 128
EMB = 64
E_RAW = 320000
E_TOT = E_RAW + N
CHUNK = 240
GRP = CHUNK // 16
P2_CHUNKS = 44
P2_PER_TILE = P2_CHUNKS * CHUNK
E_PAD = 32 * P2_PER_TILE
SLICE = NPAD // 16
AB = 2 * N + 32
BR = 1024


def _tc1_body(x_ref, wd_ref, bd_ref, wg_ref, att2_ref, hp_ref, ab_ref, c_ref):
    h = lax.dot_general(x_ref[...], wd_ref[...], (((1,), (1,)), ((), ())),
                        preferred_element_type=jnp.float32)
    h = jnp.maximum(h + bd_ref[...], 0.0)
    hp = lax.dot_general(h, wg_ref[...], (((1,), (1,)), ((), ())),
                         preferred_element_type=jnp.float32)
    hp_ref[...] = hp
    ab = lax.dot_general(hp, att2_ref[...], (((1,), (1,)), ((), ())),
                         preferred_element_type=jnp.float32)
    ab_ref[...] = ab
    m = jnp.max(ab, axis=0)
    c = m[0] + m[1]
    c = jnp.where(c >= 0.0, c, 0.2 * c)
    c_ref[...] = jnp.full((1, 16), c, jnp.float32)


def _sc_body(src_h, dst_h, ab_h, hp_h, c_h, z1_h, z2_h, out_e, out_d,
             ab_v, r0_v, r1_v, w0_v, w1_v, s0_v, s1_v, d0_v, d1_v,
             e0_v, e1_v, p0_v, p1_v, cv_v, denom_s, embed_s,
             g0, g1, x0, x1):
    cid = lax.axis_index("c")
    sid = lax.axis_index("s")
    wid = cid * 16 + sid
    base = wid * P2_PER_TILE

    pltpu.sync_copy(ab_h, ab_v)
    pltpu.sync_copy(c_h.at[0], cv_v)
    pltpu.sync_copy(z1_h, denom_s.at[pl.ds(sid * SLICE, SLICE)])
    pltpu.sync_copy(z2_h, embed_s.at[pl.ds(sid * SLICE, SLICE)])
    plsc.subcore_barrier()

    cvec = cv_v[...]
    slots = ((s0_v, d0_v, e0_v, r0_v, w0_v, p0_v, g0, x0),
             (s1_v, d1_v, e1_v, r1_v, w1_v, p1_v, g1, x1))
    iota16 = lax.iota(jnp.int32, 16)

    def drain_scatters(k):
        _, _, sdid, _, srows, pb, _, ssem = slots[k]
        pltpu.make_async_copy(srows, embed_s.at[sdid], ssem).wait()
        pltpu.make_async_copy(pb, denom_s.at[sdid], ssem).wait()

    def fetch(i, k):
        sidx, didx, _, rows, _, _, gsem, _ = slots[k]
        off = base + i * CHUNK
        pltpu.sync_copy(src_h.at[pl.ds(off, CHUNK)], sidx)
        pltpu.sync_copy(dst_h.at[pl.ds(off, CHUNK)], didx)
        pltpu.async_copy(hp_h.at[sidx], rows, gsem)

    def process(i, k):
        sidx, didx, sdid, rows, srows, pb, gsem, ssem = slots[k]
        pltpu.make_async_copy(hp_h.at[sidx], rows, gsem).wait()

        def grp(g, c2):
            si = sidx[pl.ds(g * 16, 16)]
            di = didx[pl.ds(g * 16, 16)]
            sdid[pl.ds(g * 16, 16)] = di
            e = (plsc.load_gather(ab_v, [si + si])
                 + plsc.load_gather(ab_v, [di + di + 1]))
            e = jnp.where(e >= 0.0, e, 0.2 * e)
            pb[pl.ds(g * 16, 16)] = jnp.exp(e - cvec)
            return c2

        lax.fori_loop(0, GRP, grp, 0)

        def scale(g, c2):
            p16 = pb[pl.ds(g * 16, 16)]
            for j in range(16):
                pe = p16.at[jnp.full((16,), j, jnp.int32)].get(
                    mode="promise_in_bounds")
                e = g * 16 + j
                for cb in range(4):
                    srows[e, pl.ds(16 * cb, 16)] = (
                        rows[e, pl.ds(16 * cb, 16)] * pe)
            return c2

        lax.fori_loop(0, GRP, scale, 0)
        pltpu.async_copy(srows, embed_s.at[sdid], ssem, add=True)
        pltpu.async_copy(pb, denom_s.at[sdid], ssem, add=True)

    fetch(0, 0)
    fetch(1, 1)

    def ring_iter(o, carry):
        for b in range(2):
            i = o * 2 + b

            @pl.when(i >= 2)
            def _():
                drain_scatters(b)

            process(i, b)

            @pl.when(i < P2_CHUNKS - 2)
            def _():
                fetch(i + 2, b)
        return carry

    lax.fori_loop(0, P2_CHUNKS // 2, ring_iter, 0)
    drain_scatters(0)
    drain_scatters(1)

    plsc.subcore_barrier()
    pltpu.sync_copy(embed_s.at[pl.ds(sid * SLICE, SLICE)],
                    out_e.at[cid, pl.ds(sid * SLICE, SLICE)])
    pltpu.sync_copy(denom_s.at[pl.ds(sid * SLICE, SLICE)],
                    out_d.at[cid, pl.ds(sid * SLICE, SLICE)])


def _emb_body(nr, dn, bg, emb_ref):
    n = nr[0] + nr[1]
    d = dn[0:1, :] + dn[1:2, :] + 1e-16
    emb_ref[...] = n / jnp.transpose(d) + bg[...]


def _tc2_body(er_ref, ec_ref, out_ref):
    logits = lax.dot_general(er_ref[...], ec_ref[...], (((1,), (1,)), ((), ())),
                             preferred_element_type=jnp.float32)
    out_ref[...] = jax.nn.sigmoid(logits)


def kernel(x, adj, W_dense, b_dense, W_gat, att_src, att_dst, b_gat):
    f32 = jnp.float32
    loop = jnp.arange(N, dtype=adj.dtype)
    pad = E_PAD - E_TOT
    src = jnp.concatenate([adj[0].astype(jnp.int32), loop.astype(jnp.int32),
                           jnp.zeros((pad,), jnp.int32)])
    dst = jnp.concatenate([adj[1].astype(jnp.int32), loop.astype(jnp.int32),
                           jnp.full((pad,), N, jnp.int32)])

    hp, ab, c_c = pl.pallas_call(
        _tc1_body,
        out_shape=[
            jax.ShapeDtypeStruct((N, EMB), f32),
            jax.ShapeDtypeStruct((N, 2), f32),
            jax.ShapeDtypeStruct((1, 16), f32),
        ],
    )(x, W_dense, b_dense.reshape(1, EMB), W_gat,
      jnp.stack([att_src, att_dst], axis=0))
    ab_flat = jnp.concatenate([ab, jnp.zeros((16, 2), f32)]).reshape(-1)

    mesh = plsc.VectorSubcoreMesh(core_axis_name="c", subcore_axis_name="s",
                                  num_cores=2, num_subcores=16)
    sc_fn = pl.kernel(
        _sc_body,
        out_type=[
            jax.ShapeDtypeStruct((2, NPAD, EMB), f32),
            jax.ShapeDtypeStruct((2, NPAD), f32),
        ],
        mesh=mesh,
        compiler_params=pltpu.CompilerParams(needs_layout_passes=False,
                                             use_tc_tiling_on_sc=False),
        scratch_types=[
            pltpu.VMEM((AB,), f32),
            pltpu.VMEM((CHUNK, EMB), f32),
            pltpu.VMEM((CHUNK, EMB), f32),
            pltpu.VMEM((CHUNK, EMB), f32),
            pltpu.VMEM((CHUNK, EMB), f32),
            pltpu.VMEM((CHUNK,), jnp.int32),
            pltpu.VMEM((CHUNK,), jnp.int32),
            pltpu.VMEM((CHUNK,), jnp.int32),
            pltpu.VMEM((CHUNK,), jnp.int32),
            pltpu.VMEM((CHUNK,), jnp.int32),
            pltpu.VMEM((CHUNK,), jnp.int32),
            pltpu.VMEM((CHUNK,), f32),
            pltpu.VMEM((CHUNK,), f32),
            pltpu.VMEM((16,), f32),
            pltpu.VMEM_SHARED((NPAD,), f32),
            pltpu.VMEM_SHARED((NPAD, EMB), f32),
            pltpu.SemaphoreType.DMA,
            pltpu.SemaphoreType.DMA,
            pltpu.SemaphoreType.DMA,
            pltpu.SemaphoreType.DMA,
        ],
    )
    z1 = jnp.zeros((SLICE,), f32)
    z2 = jnp.zeros((SLICE, EMB), f32)
    nums, dens = sc_fn(src, dst, ab_flat, hp, c_c, z1, z2)

    emb_full = pl.pallas_call(
        _emb_body,
        out_shape=jax.ShapeDtypeStruct((NPAD, EMB), f32),
    )(nums, dens, b_gat.reshape(1, EMB))

    out = pl.pallas_call(
        _tc2_body,
        grid=(NPAD // BR, NPAD // BR),
        in_specs=[
            pl.BlockSpec((BR, EMB), lambda i, j: (i, 0)),
            pl.BlockSpec((BR, EMB), lambda i, j: (j, 0)),
        ],
        out_specs=pl.BlockSpec((BR, BR), lambda i, j: (i, j)),
        out_shape=jax.ShapeDtypeStruct((N, N), f32),
    )(emb_full, emb_full)
    return (out, emb_full[:N])


# speedup vs baseline: 22.2702x; 1.0063x over previous
import functools

import jax
import jax.numpy as jnp
from jax import lax
from jax.experimental import pallas as pl
from jax.experimental.pallas import tpu as pltpu
from jax.experimental.pallas import tpu_sc as plsc

N = 10000
NPAD = 10240
IN_DIM = 128
EMB = 64
E_RAW = 320000
E_TOT = E_RAW + N
CHUNK = 176
GRP = CHUNK // 16
P2_CHUNKS = 60
P2_PER_TILE = P2_CHUNKS * CHUNK
E_PAD = 32 * P2_PER_TILE
SLICE = NPAD // 16
AB = 2 * N + 32
BR = 1024


def _tc1_body(x_ref, wd_ref, bd_ref, wg_ref, att2_ref, hp_ref, ab_ref, c_ref):
    h = lax.dot_general(x_ref[...], wd_ref[...], (((1,), (1,)), ((), ())),
                        preferred_element_type=jnp.float32)
    h = jnp.maximum(h + bd_ref[...], 0.0)
    hp = lax.dot_general(h, wg_ref[...], (((1,), (1,)), ((), ())),
                         preferred_element_type=jnp.float32)
    hp_ref[...] = hp
    ab = lax.dot_general(hp, att2_ref[...], (((1,), (1,)), ((), ())),
                         preferred_element_type=jnp.float32)
    ab_ref[...] = ab
    m = jnp.max(ab, axis=0)
    c = m[0] + m[1]
    c = jnp.where(c >= 0.0, c, 0.2 * c)
    c_ref[...] = jnp.full((1, 16), c, jnp.float32)


def _sc_body(src_h, dst_h, ab_h, hp_h, c_h, z1_h, z2_h, out_e, out_d,
             ab_v, sall_v, dall_v, r0_v, r1_v, w0_v, w1_v,
             e0_v, e1_v, p0_v, p1_v, cv_v, denom_s, embed_s,
             g0, g1, x0, x1):
    cid = lax.axis_index("c")
    sid = lax.axis_index("s")
    wid = cid * 16 + sid
    base = wid * P2_PER_TILE

    pltpu.sync_copy(ab_h, ab_v)
    pltpu.sync_copy(c_h.at[0], cv_v)
    pltpu.sync_copy(src_h.at[pl.ds(base, P2_PER_TILE)], sall_v)
    pltpu.sync_copy(dst_h.at[pl.ds(base, P2_PER_TILE)], dall_v)
    pltpu.sync_copy(z1_h, denom_s.at[pl.ds(sid * SLICE, SLICE)])
    pltpu.sync_copy(z2_h, embed_s.at[pl.ds(sid * SLICE, SLICE)])
    plsc.subcore_barrier()

    cvec = cv_v[...]
    slots = ((e0_v, r0_v, w0_v, p0_v, g0, x0),
             (e1_v, r1_v, w1_v, p1_v, g1, x1))

    def drain_scatters(k):
        sdid, _, srows, pb, _, ssem = slots[k]
        pltpu.make_async_copy(srows, embed_s.at[sdid], ssem).wait()
        pltpu.make_async_copy(pb, denom_s.at[sdid], ssem).wait()

    def fetch(i, k):
        _, rows, _, _, gsem, _ = slots[k]
        pltpu.async_copy(hp_h.at[sall_v.at[pl.ds(i * CHUNK, CHUNK)]],
                         rows, gsem)

    def process(i, k):
        sdid, rows, srows, pb, gsem, ssem = slots[k]
        pltpu.make_async_copy(hp_h.at[sall_v.at[pl.ds(i * CHUNK, CHUNK)]],
                              rows, gsem).wait()
        ioff = i * CHUNK

        def grp(g, c2):
            si = sall_v[pl.ds(ioff + g * 16, 16)]
            di = dall_v[pl.ds(ioff + g * 16, 16)]
            sdid[pl.ds(g * 16, 16)] = di
            e = (plsc.load_gather(ab_v, [si + si])
                 + plsc.load_gather(ab_v, [di + di + 1]))
            e = jnp.where(e >= 0.0, e, 0.2 * e)
            pb[pl.ds(g * 16, 16)] = jnp.exp(e - cvec)
            return c2

        lax.fori_loop(0, GRP, grp, 0)

        def scale(g, c2):
            p16 = pb[pl.ds(g * 16, 16)]
            for j in range(16):
                pe = p16.at[jnp.full((16,), j, jnp.int32)].get(
                    mode="promise_in_bounds")
                e = g * 16 + j
                for cb in range(4):
                    srows[e, pl.ds(16 * cb, 16)] = (
                        rows[e, pl.ds(16 * cb, 16)] * pe)
            return c2

        lax.fori_loop(0, GRP, scale, 0)
        pltpu.async_copy(srows, embed_s.at[sdid], ssem, add=True)
        pltpu.async_copy(pb, denom_s.at[sdid], ssem, add=True)

    fetch(0, 0)
    fetch(1, 1)

    def ring_iter(o, carry):
        for b in range(2):
            i = o * 2 + b

            @pl.when(i >= 2)
            def _():
                drain_scatters(b)

            process(i, b)

            @pl.when(i < P2_CHUNKS - 2)
            def _():
                fetch(i + 2, b)
        return carry

    lax.fori_loop(0, P2_CHUNKS // 2, ring_iter, 0)
    drain_scatters(0)
    drain_scatters(1)

    plsc.subcore_barrier()
    pltpu.sync_copy(embed_s.at[pl.ds(sid * SLICE, SLICE)],
                    out_e.at[cid, pl.ds(sid * SLICE, SLICE)])
    pltpu.sync_copy(denom_s.at[pl.ds(sid * SLICE, SLICE)],
                    out_d.at[cid, pl.ds(sid * SLICE, SLICE)])


def _emb_body(nr, dn, bg, emb_ref):
    n = nr[0] + nr[1]
    d = dn[0:1, :] + dn[1:2, :] + 1e-16
    emb_ref[...] = n / jnp.transpose(d) + bg[...]


def _tc2_body(er_ref, ec_ref, out_ref):
    logits = lax.dot_general(er_ref[...], ec_ref[...], (((1,), (1,)), ((), ())),
                             preferred_element_type=jnp.float32)
    out_ref[...] = jax.nn.sigmoid(logits)


def kernel(x, adj, W_dense, b_dense, W_gat, att_src, att_dst, b_gat):
    f32 = jnp.float32
    loop = jnp.arange(N, dtype=adj.dtype)
    pad = E_PAD - E_TOT
    src = jnp.concatenate([adj[0].astype(jnp.int32), loop.astype(jnp.int32),
                           jnp.zeros((pad,), jnp.int32)])
    dst = jnp.concatenate([adj[1].astype(jnp.int32), loop.astype(jnp.int32),
                           jnp.full((pad,), N, jnp.int32)])

    hp, ab, c_c = pl.pallas_call(
        _tc1_body,
        out_shape=[
            jax.ShapeDtypeStruct((N, EMB), f32),
            jax.ShapeDtypeStruct((N, 2), f32),
            jax.ShapeDtypeStruct((1, 16), f32),
        ],
    )(x, W_dense, b_dense.reshape(1, EMB), W_gat,
      jnp.stack([att_src, att_dst], axis=0))
    ab_flat = jnp.concatenate([ab, jnp.zeros((16, 2), f32)]).reshape(-1)

    mesh = plsc.VectorSubcoreMesh(core_axis_name="c", subcore_axis_name="s",
                                  num_cores=2, num_subcores=16)
    sc_fn = pl.kernel(
        _sc_body,
        out_type=[
            jax.ShapeDtypeStruct((2, NPAD, EMB), f32),
            jax.ShapeDtypeStruct((2, NPAD), f32),
        ],
        mesh=mesh,
        compiler_params=pltpu.CompilerParams(needs_layout_passes=False,
                                             use_tc_tiling_on_sc=False),
        scratch_types=[
            pltpu.VMEM((AB,), f32),
            pltpu.VMEM((P2_PER_TILE,), jnp.int32),
            pltpu.VMEM((P2_PER_TILE,), jnp.int32),
            pltpu.VMEM((CHUNK, EMB), f32),
            pltpu.VMEM((CHUNK, EMB), f32),
            pltpu.VMEM((CHUNK, EMB), f32),
            pltpu.VMEM((CHUNK, EMB), f32),
            pltpu.VMEM((CHUNK,), jnp.int32),
            pltpu.VMEM((CHUNK,), jnp.int32),
            pltpu.VMEM((CHUNK,), f32),
            pltpu.VMEM((CHUNK,), f32),
            pltpu.VMEM((16,), f32),
            pltpu.VMEM_SHARED((NPAD,), f32),
            pltpu.VMEM_SHARED((NPAD, EMB), f32),
            pltpu.SemaphoreType.DMA,
            pltpu.SemaphoreType.DMA,
            pltpu.SemaphoreType.DMA,
            pltpu.SemaphoreType.DMA,
        ],
    )
    z1 = jnp.zeros((SLICE,), f32)
    z2 = jnp.zeros((SLICE, EMB), f32)
    nums, dens = sc_fn(src, dst, ab_flat, hp, c_c, z1, z2)

    emb_full = pl.pallas_call(
        _emb_body,
        out_shape=jax.ShapeDtypeStruct((NPAD, EMB), f32),
    )(nums, dens, b_gat.reshape(1, EMB))

    out = pl.pallas_call(
        _tc2_body,
        grid=(NPAD // BR, NPAD // BR),
        in_specs=[
            pl.BlockSpec((BR, EMB), lambda i, j: (i, 0)),
            pl.BlockSpec((BR, EMB), lambda i, j: (j, 0)),
        ],
        out_specs=pl.BlockSpec((BR, BR), lambda i, j: (i, j)),
        out_shape=jax.ShapeDtypeStruct((N, N), f32),
    )(emb_full, emb_full)
    return (out, emb_full[:N])


# speedup vs baseline: 30.6976x; 1.3784x over previous
import functools

import jax
import jax.numpy as jnp
from jax import lax
from jax.experimental import pallas as pl
from jax.experimental.pallas import tpu as pltpu
from jax.experimental.pallas import tpu_sc as plsc

N = 10000
NPAD = 10240
IN_DIM = 128
EMB = 64
E_RAW = 320000
E_TOT = E_RAW + N
CHUNK = 96
GRP = CHUNK // 16
P2_CHUNKS = 110
P2_PER_TILE = P2_CHUNKS * CHUNK
E_PAD = 32 * P2_PER_TILE
SLICE = NPAD // 16
AB = 2 * N + 32
BR = 1024


def _tc1_body(x_ref, wd_ref, bd_ref, wg_ref, att2_ref, hp_ref, ab_ref, c_ref):
    h = lax.dot_general(x_ref[...], wd_ref[...], (((1,), (1,)), ((), ())),
                        preferred_element_type=jnp.float32)
    h = jnp.maximum(h + bd_ref[...], 0.0)
    hp = lax.dot_general(h, wg_ref[...], (((1,), (1,)), ((), ())),
                         preferred_element_type=jnp.float32)
    hp_ref[...] = hp
    ab = lax.dot_general(hp, att2_ref[...], (((1,), (1,)), ((), ())),
                         preferred_element_type=jnp.float32)
    ab_ref[...] = ab
    m = jnp.max(ab, axis=0)
    c = m[0] + m[1]
    c = jnp.where(c >= 0.0, c, 0.2 * c)
    c_ref[...] = jnp.full((1, 16), c, jnp.float32)


def _sc_body(src_h, dst_h, ab_h, hp_h, c_h, z1_h, z2_h, out_e, out_d,
             ab_v, r0_v, r1_v, w0_v, w1_v, s0_v, s1_v, d0_v, d1_v,
             e0_v, e1_v, p0_v, p1_v, cv_v, denom_s, embed_s, hp_s,
             g0, g1, x0, x1, i0, i1):
    cid = lax.axis_index("c")
    sid = lax.axis_index("s")
    wid = cid * 16 + sid
    base = wid * P2_PER_TILE

    pltpu.sync_copy(ab_h, ab_v)
    pltpu.sync_copy(c_h.at[0], cv_v)
    pltpu.sync_copy(hp_h.at[pl.ds(sid * (N // 16), N // 16)],
                    hp_s.at[pl.ds(sid * (N // 16), N // 16)])
    pltpu.sync_copy(z1_h, denom_s.at[pl.ds(sid * SLICE, SLICE)])
    pltpu.sync_copy(z2_h, embed_s.at[pl.ds(sid * SLICE, SLICE)])
    plsc.subcore_barrier()

    cvec = cv_v[...]
    slots = ((s0_v, d0_v, e0_v, r0_v, w0_v, p0_v, g0, x0, i0),
             (s1_v, d1_v, e1_v, r1_v, w1_v, p1_v, g1, x1, i1))

    def drain_scatters(k):
        _, _, sdid, _, srows, pb, _, ssem, _ = slots[k]
        pltpu.make_async_copy(srows, embed_s.at[sdid], ssem).wait()
        pltpu.make_async_copy(pb, denom_s.at[sdid], ssem).wait()

    def fetch_idx(i, k):
        sidx, didx, _, _, _, _, _, _, isem = slots[k]
        off = base + i * CHUNK
        pltpu.async_copy(src_h.at[pl.ds(off, CHUNK)], sidx, isem)
        pltpu.async_copy(dst_h.at[pl.ds(off, CHUNK)], didx, isem)

    def wait_idx(k):
        sidx, didx, _, _, _, _, _, _, isem = slots[k]
        pltpu.make_async_copy(src_h.at[pl.ds(0, CHUNK)], sidx, isem).wait()
        pltpu.make_async_copy(dst_h.at[pl.ds(0, CHUNK)], didx, isem).wait()

    def fetch_rows(k):
        sidx, _, _, rows, _, _, gsem, _, _ = slots[k]
        pltpu.async_copy(hp_s.at[sidx], rows, gsem)

    def process(i, k):
        sidx, didx, sdid, rows, srows, pb, gsem, ssem, _ = slots[k]
        pltpu.make_async_copy(hp_s.at[sidx], rows, gsem).wait()

        @pl.when(i < P2_CHUNKS - 1)
        def _():
            wait_idx(k ^ 1)
            fetch_rows(k ^ 1)

        @pl.when(i >= 2)
        def _():
            drain_scatters(k)

        def grp(g, c2):
            si = sidx[pl.ds(g * 16, 16)]
            di = didx[pl.ds(g * 16, 16)]
            sdid[pl.ds(g * 16, 16)] = di
            e = (plsc.load_gather(ab_v, [si + si])
                 + plsc.load_gather(ab_v, [di + di + 1]))
            e = jnp.where(e >= 0.0, e, 0.2 * e)
            pb[pl.ds(g * 16, 16)] = jnp.exp(e - cvec)
            return c2

        lax.fori_loop(0, GRP, grp, 0)

        def scale(g, c2):
            p16 = pb[pl.ds(g * 16, 16)]
            for j in range(16):
                pe = p16.at[jnp.full((16,), j, jnp.int32)].get(
                    mode="promise_in_bounds")
                e = g * 16 + j
                for cb in range(4):
                    srows[e, pl.ds(16 * cb, 16)] = (
                        rows[e, pl.ds(16 * cb, 16)] * pe)
            return c2

        lax.fori_loop(0, GRP, scale, 0)
        pltpu.async_copy(srows, embed_s.at[sdid], ssem, add=True)
        pltpu.async_copy(pb, denom_s.at[sdid], ssem, add=True)

        @pl.when(i < P2_CHUNKS - 2)
        def _():
            fetch_idx(i + 2, k)

    fetch_idx(0, 0)
    fetch_idx(1, 1)
    wait_idx(0)
    fetch_rows(0)

    def ring_iter(o, carry):
        process(o * 2, 0)
        process(o * 2 + 1, 1)
        return carry

    lax.fori_loop(0, P2_CHUNKS // 2, ring_iter, 0)
    drain_scatters(0)
    drain_scatters(1)

    plsc.subcore_barrier()
    pltpu.sync_copy(embed_s.at[pl.ds(sid * SLICE, SLICE)],
                    out_e.at[cid, pl.ds(sid * SLICE, SLICE)])
    pltpu.sync_copy(denom_s.at[pl.ds(sid * SLICE, SLICE)],
                    out_d.at[cid, pl.ds(sid * SLICE, SLICE)])


def _emb_body(nr, dn, bg, emb_ref):
    n = nr[0] + nr[1]
    d = dn[0:1, :] + dn[1:2, :] + 1e-16
    emb_ref[...] = n / jnp.transpose(d) + bg[...]


def _tc2_body(er_ref, ec_ref, out_ref):
    logits = lax.dot_general(er_ref[...], ec_ref[...], (((1,), (1,)), ((), ())),
                             preferred_element_type=jnp.float32)
    out_ref[...] = jax.nn.sigmoid(logits)


def kernel(x, adj, W_dense, b_dense, W_gat, att_src, att_dst, b_gat):
    f32 = jnp.float32
    loop = jnp.arange(N, dtype=adj.dtype)
    pad = E_PAD - E_TOT
    src = jnp.concatenate([adj[0].astype(jnp.int32), loop.astype(jnp.int32),
                           jnp.zeros((pad,), jnp.int32)])
    dst = jnp.concatenate([adj[1].astype(jnp.int32), loop.astype(jnp.int32),
                           jnp.full((pad,), N, jnp.int32)])

    hp, ab, c_c = pl.pallas_call(
        _tc1_body,
        out_shape=[
            jax.ShapeDtypeStruct((N, EMB), f32),
            jax.ShapeDtypeStruct((N, 2), f32),
            jax.ShapeDtypeStruct((1, 16), f32),
        ],
    )(x, W_dense, b_dense.reshape(1, EMB), W_gat,
      jnp.stack([att_src, att_dst], axis=0))
    ab_flat = jnp.concatenate([ab, jnp.zeros((16, 2), f32)]).reshape(-1)

    mesh = plsc.VectorSubcoreMesh(core_axis_name="c", subcore_axis_name="s",
                                  num_cores=2, num_subcores=16)
    sc_fn = pl.kernel(
        _sc_body,
        out_type=[
            jax.ShapeDtypeStruct((2, NPAD, EMB), f32),
            jax.ShapeDtypeStruct((2, NPAD), f32),
        ],
        mesh=mesh,
        compiler_params=pltpu.CompilerParams(needs_layout_passes=False,
                                             use_tc_tiling_on_sc=False),
        scratch_types=[
            pltpu.VMEM((AB,), f32),
            pltpu.VMEM((CHUNK, EMB), f32),
            pltpu.VMEM((CHUNK, EMB), f32),
            pltpu.VMEM((CHUNK, EMB), f32),
            pltpu.VMEM((CHUNK, EMB), f32),
            pltpu.VMEM((CHUNK,), jnp.int32),
            pltpu.VMEM((CHUNK,), jnp.int32),
            pltpu.VMEM((CHUNK,), jnp.int32),
            pltpu.VMEM((CHUNK,), jnp.int32),
            pltpu.VMEM((CHUNK,), jnp.int32),
            pltpu.VMEM((CHUNK,), jnp.int32),
            pltpu.VMEM((CHUNK,), f32),
            pltpu.VMEM((CHUNK,), f32),
            pltpu.VMEM((16,), f32),
            pltpu.VMEM_SHARED((NPAD,), f32),
            pltpu.VMEM_SHARED((NPAD, EMB), f32),
            pltpu.VMEM_SHARED((N, EMB), f32),
            pltpu.SemaphoreType.DMA,
            pltpu.SemaphoreType.DMA,
            pltpu.SemaphoreType.DMA,
            pltpu.SemaphoreType.DMA,
            pltpu.SemaphoreType.DMA,
            pltpu.SemaphoreType.DMA,
        ],
    )
    z1 = jnp.zeros((SLICE,), f32)
    z2 = jnp.zeros((SLICE, EMB), f32)
    nums, dens = sc_fn(src, dst, ab_flat, hp, c_c, z1, z2)

    emb_full = pl.pallas_call(
        _emb_body,
        out_shape=jax.ShapeDtypeStruct((NPAD, EMB), f32),
    )(nums, dens, b_gat.reshape(1, EMB))

    out = pl.pallas_call(
        _tc2_body,
        grid=(NPAD // BR, NPAD // BR),
        in_specs=[
            pl.BlockSpec((BR, EMB), lambda i, j: (i, 0)),
            pl.BlockSpec((BR, EMB), lambda i, j: (j, 0)),
        ],
        out_specs=pl.BlockSpec((BR, BR), lambda i, j: (i, j)),
        out_shape=jax.ShapeDtypeStruct((N, N), f32),
    )(emb_full, emb_full)
    return (out, emb_full[:N])


# speedup vs baseline: 34.3788x; 1.1199x over previous
import functools

import jax
import jax.numpy as jnp
from jax import lax
from jax.experimental import pallas as pl
from jax.experimental.pallas import tpu as pltpu
from jax.experimental.pallas import tpu_sc as plsc

N = 10000
NPAD = 10240
IN_DIM = 128
EMB = 64
E_RAW = 320000
E_TOT = E_RAW + N
CHUNK = 96
GRP = CHUNK // 16
P2_CHUNKS = 110
P2_PER_TILE = P2_CHUNKS * CHUNK
E_PAD = 32 * P2_PER_TILE
SLICE = NPAD // 16
AB = 2 * N + 32
BR = 2048


def _tc1_body(x_ref, wd_ref, bd_ref, wg_ref, att2_ref, hp_ref, ab_ref, c_ref):
    h = lax.dot_general(x_ref[...], wd_ref[...], (((1,), (1,)), ((), ())),
                        preferred_element_type=jnp.float32)
    h = jnp.maximum(h + bd_ref[...], 0.0)
    hp = lax.dot_general(h, wg_ref[...], (((1,), (1,)), ((), ())),
                         preferred_element_type=jnp.float32)
    hp_ref[...] = hp
    ab = lax.dot_general(hp, att2_ref[...], (((1,), (1,)), ((), ())),
                         preferred_element_type=jnp.float32)
    ab_ref[...] = ab
    m = jnp.max(ab, axis=0)
    c = m[0] + m[1]
    c = jnp.where(c >= 0.0, c, 0.2 * c)
    c_ref[...] = jnp.full((1, 16), c, jnp.float32)


def _sc_body(src_h, dst_h, ab_h, hp_h, c_h, z1_h, z2_h, out_e, out_d,
             ab_v, r0_v, r1_v, w0_v, w1_v, s0_v, s1_v, d0_v, d1_v,
             e0_v, e1_v, p0_v, p1_v, cv_v, denom_s, embed_s, hp_s,
             g0, g1, x0, x1, i0, i1):
    cid = lax.axis_index("c")
    sid = lax.axis_index("s")
    wid = cid * 16 + sid
    base = wid * P2_PER_TILE

    pltpu.sync_copy(ab_h, ab_v)
    pltpu.sync_copy(c_h.at[0], cv_v)
    pltpu.sync_copy(hp_h.at[pl.ds(sid * (N // 16), N // 16)],
                    hp_s.at[pl.ds(sid * (N // 16), N // 16)])
    pltpu.sync_copy(z1_h, denom_s.at[pl.ds(sid * SLICE, SLICE)])
    pltpu.sync_copy(z2_h, embed_s.at[pl.ds(sid * SLICE, SLICE)])
    plsc.subcore_barrier()

    cvec = cv_v[...]
    slots = ((s0_v, d0_v, e0_v, r0_v, w0_v, p0_v, g0, x0, i0),
             (s1_v, d1_v, e1_v, r1_v, w1_v, p1_v, g1, x1, i1))

    def drain_scatters(k):
        _, _, sdid, _, srows, pb, _, ssem, _ = slots[k]
        pltpu.make_async_copy(srows, embed_s.at[sdid], ssem).wait()
        pltpu.make_async_copy(pb, denom_s.at[sdid], ssem).wait()

    def fetch_idx(i, k):
        sidx, didx, _, _, _, _, _, _, isem = slots[k]
        off = base + i * CHUNK
        pltpu.async_copy(src_h.at[pl.ds(off, CHUNK)], sidx, isem)
        pltpu.async_copy(dst_h.at[pl.ds(off, CHUNK)], didx, isem)

    def wait_idx(k):
        sidx, didx, _, _, _, _, _, _, isem = slots[k]
        pltpu.make_async_copy(src_h.at[pl.ds(0, CHUNK)], sidx, isem).wait()
        pltpu.make_async_copy(dst_h.at[pl.ds(0, CHUNK)], didx, isem).wait()

    def fetch_rows(k):
        sidx, _, _, rows, _, _, gsem, _, _ = slots[k]
        pltpu.async_copy(hp_s.at[sidx], rows, gsem)

    def process(i, k):
        sidx, didx, sdid, rows, srows, pb, gsem, ssem, _ = slots[k]
        pltpu.make_async_copy(hp_s.at[sidx], rows, gsem).wait()

        @pl.when(i < P2_CHUNKS - 1)
        def _():
            wait_idx(k ^ 1)
            fetch_rows(k ^ 1)

        @pl.when(i >= 2)
        def _():
            drain_scatters(k)

        def grp(g, c2):
            si = sidx[pl.ds(g * 16, 16)]
            di = didx[pl.ds(g * 16, 16)]
            sdid[pl.ds(g * 16, 16)] = di
            e = (plsc.load_gather(ab_v, [si + si])
                 + plsc.load_gather(ab_v, [di + di + 1]))
            e = jnp.where(e >= 0.0, e, 0.2 * e)
            pb[pl.ds(g * 16, 16)] = jnp.exp(e - cvec)
            return c2

        lax.fori_loop(0, GRP, grp, 0)

        def scale(g, c2):
            p16 = pb[pl.ds(g * 16, 16)]
            for j in range(16):
                pe = p16.at[jnp.full((16,), j, jnp.int32)].get(
                    mode="promise_in_bounds")
                e = g * 16 + j
                for cb in range(4):
                    srows[e, pl.ds(16 * cb, 16)] = (
                        rows[e, pl.ds(16 * cb, 16)] * pe)
            return c2

        lax.fori_loop(0, GRP, scale, 0)
        pltpu.async_copy(srows, embed_s.at[sdid], ssem, add=True)
        pltpu.async_copy(pb, denom_s.at[sdid], ssem, add=True)

        @pl.when(i < P2_CHUNKS - 2)
        def _():
            fetch_idx(i + 2, k)

    fetch_idx(0, 0)
    fetch_idx(1, 1)
    wait_idx(0)
    fetch_rows(0)

    def ring_iter(o, carry):
        process(o * 2, 0)
        process(o * 2 + 1, 1)
        return carry

    lax.fori_loop(0, P2_CHUNKS // 2, ring_iter, 0)
    drain_scatters(0)
    drain_scatters(1)

    plsc.subcore_barrier()
    pltpu.sync_copy(embed_s.at[pl.ds(sid * SLICE, SLICE)],
                    out_e.at[cid, pl.ds(sid * SLICE, SLICE)])
    pltpu.sync_copy(denom_s.at[pl.ds(sid * SLICE, SLICE)],
                    out_d.at[cid, pl.ds(sid * SLICE, SLICE)])


def _emb_body(nr, dn, bg, emb_ref):
    n = nr[0] + nr[1]
    d = dn[0:1, :] + dn[1:2, :] + 1e-16
    emb_ref[...] = n / jnp.transpose(d) + bg[...]


def _tc2_body(er_ref, ec_ref, out_ref):
    logits = lax.dot_general(er_ref[...], ec_ref[...], (((1,), (1,)), ((), ())),
                             preferred_element_type=jnp.float32)
    out_ref[...] = jax.nn.sigmoid(logits)


def kernel(x, adj, W_dense, b_dense, W_gat, att_src, att_dst, b_gat):
    f32 = jnp.float32
    loop = jnp.arange(N, dtype=adj.dtype)
    pad = E_PAD - E_TOT
    src = jnp.concatenate([adj[0].astype(jnp.int32), loop.astype(jnp.int32),
                           jnp.zeros((pad,), jnp.int32)])
    dst = jnp.concatenate([adj[1].astype(jnp.int32), loop.astype(jnp.int32),
                           jnp.full((pad,), N, jnp.int32)])

    hp, ab, c_c = pl.pallas_call(
        _tc1_body,
        out_shape=[
            jax.ShapeDtypeStruct((N, EMB), f32),
            jax.ShapeDtypeStruct((N, 2), f32),
            jax.ShapeDtypeStruct((1, 16), f32),
        ],
    )(x, W_dense, b_dense.reshape(1, EMB), W_gat,
      jnp.stack([att_src, att_dst], axis=0))
    ab_flat = jnp.concatenate([ab, jnp.zeros((16, 2), f32)]).reshape(-1)

    mesh = plsc.VectorSubcoreMesh(core_axis_name="c", subcore_axis_name="s",
                                  num_cores=2, num_subcores=16)
    sc_fn = pl.kernel(
        _sc_body,
        out_type=[
            jax.ShapeDtypeStruct((2, NPAD, EMB), f32),
            jax.ShapeDtypeStruct((2, NPAD), f32),
        ],
        mesh=mesh,
        compiler_params=pltpu.CompilerParams(needs_layout_passes=False,
                                             use_tc_tiling_on_sc=False),
        scratch_types=[
            pltpu.VMEM((AB,), f32),
            pltpu.VMEM((CHUNK, EMB), f32),
            pltpu.VMEM((CHUNK, EMB), f32),
            pltpu.VMEM((CHUNK, EMB), f32),
            pltpu.VMEM((CHUNK, EMB), f32),
            pltpu.VMEM((CHUNK,), jnp.int32),
            pltpu.VMEM((CHUNK,), jnp.int32),
            pltpu.VMEM((CHUNK,), jnp.int32),
            pltpu.VMEM((CHUNK,), jnp.int32),
            pltpu.VMEM((CHUNK,), jnp.int32),
            pltpu.VMEM((CHUNK,), jnp.int32),
            pltpu.VMEM((CHUNK,), f32),
            pltpu.VMEM((CHUNK,), f32),
            pltpu.VMEM((16,), f32),
            pltpu.VMEM_SHARED((NPAD,), f32),
            pltpu.VMEM_SHARED((NPAD, EMB), f32),
            pltpu.VMEM_SHARED((N, EMB), f32),
            pltpu.SemaphoreType.DMA,
            pltpu.SemaphoreType.DMA,
            pltpu.SemaphoreType.DMA,
            pltpu.SemaphoreType.DMA,
            pltpu.SemaphoreType.DMA,
            pltpu.SemaphoreType.DMA,
        ],
    )
    z1 = jnp.zeros((SLICE,), f32)
    z2 = jnp.zeros((SLICE, EMB), f32)
    nums, dens = sc_fn(src, dst, ab_flat, hp, c_c, z1, z2)

    emb_full = pl.pallas_call(
        _emb_body,
        out_shape=jax.ShapeDtypeStruct((NPAD, EMB), f32),
    )(nums, dens, b_gat.reshape(1, EMB))

    out = pl.pallas_call(
        _tc2_body,
        grid=(NPAD // BR, NPAD // BR),
        in_specs=[
            pl.BlockSpec((BR, EMB), lambda i, j: (i, 0)),
            pl.BlockSpec((BR, EMB), lambda i, j: (j, 0)),
        ],
        out_specs=pl.BlockSpec((BR, BR), lambda i, j: (i, j)),
        out_shape=jax.ShapeDtypeStruct((N, N), f32),
    )(emb_full, emb_full)
    return (out, emb_full[:N])


# speedup vs baseline: 34.5366x; 1.0046x over previous
import functools

import jax
import jax.numpy as jnp
from jax import lax
from jax.experimental import pallas as pl
from jax.experimental.pallas import tpu as pltpu
from jax.experimental.pallas import tpu_sc as plsc

N = 10000
NPAD = 10240
IN_DIM = 128
EMB = 64
E_RAW = 320000
E_TOT = E_RAW + N
CHUNK = 96
GRP = CHUNK // 16
P2_CHUNKS = 110
P2_PER_TILE = P2_CHUNKS * CHUNK
E_PAD = 32 * P2_PER_TILE
SLICE = NPAD // 16
AB = 2 * N + 32
BR = 2560


def _tc1_body(x_ref, wd_ref, bd_ref, wg_ref, att2_ref, hp_ref, ab_ref, c_ref):
    h = lax.dot_general(x_ref[...], wd_ref[...], (((1,), (1,)), ((), ())),
                        preferred_element_type=jnp.float32)
    h = jnp.maximum(h + bd_ref[...], 0.0)
    hp = lax.dot_general(h, wg_ref[...], (((1,), (1,)), ((), ())),
                         preferred_element_type=jnp.float32)
    hp_ref[...] = hp
    ab = lax.dot_general(hp, att2_ref[...], (((1,), (1,)), ((), ())),
                         preferred_element_type=jnp.float32)
    ab_ref[...] = ab
    m = jnp.max(ab, axis=0)
    c = m[0] + m[1]
    c = jnp.where(c >= 0.0, c, 0.2 * c)
    c_ref[...] = jnp.full((1, 16), c, jnp.float32)


def _sc_body(src_h, dst_h, ab_h, hp_h, c_h, z1_h, z2_h, out_e, out_d,
             ab_v, r0_v, r1_v, w0_v, w1_v, s0_v, s1_v, d0_v, d1_v,
             e0_v, e1_v, p0_v, p1_v, cv_v, denom_s, embed_s, hp_s,
             g0, g1, x0, x1, i0, i1):
    cid = lax.axis_index("c")
    sid = lax.axis_index("s")
    wid = cid * 16 + sid
    base = wid * P2_PER_TILE

    pltpu.sync_copy(ab_h, ab_v)
    pltpu.sync_copy(c_h.at[0], cv_v)
    pltpu.sync_copy(hp_h.at[pl.ds(sid * (N // 16), N // 16)],
                    hp_s.at[pl.ds(sid * (N // 16), N // 16)])
    pltpu.sync_copy(z1_h, denom_s.at[pl.ds(sid * SLICE, SLICE)])
    pltpu.sync_copy(z2_h, embed_s.at[pl.ds(sid * SLICE, SLICE)])
    plsc.subcore_barrier()

    cvec = cv_v[...]
    slots = ((s0_v, d0_v, e0_v, r0_v, w0_v, p0_v, g0, x0, i0),
             (s1_v, d1_v, e1_v, r1_v, w1_v, p1_v, g1, x1, i1))

    def drain_scatters(k):
        _, _, sdid, _, srows, pb, _, ssem, _ = slots[k]
        pltpu.make_async_copy(srows, embed_s.at[sdid], ssem).wait()
        pltpu.make_async_copy(pb, denom_s.at[sdid], ssem).wait()

    def fetch_idx(i, k):
        sidx, didx, _, _, _, _, _, _, isem = slots[k]
        off = base + i * CHUNK
        pltpu.async_copy(src_h.at[pl.ds(off, CHUNK)], sidx, isem)
        pltpu.async_copy(dst_h.at[pl.ds(off, CHUNK)], didx, isem)

    def wait_idx(k):
        sidx, didx, _, _, _, _, _, _, isem = slots[k]
        pltpu.make_async_copy(src_h.at[pl.ds(0, CHUNK)], sidx, isem).wait()
        pltpu.make_async_copy(dst_h.at[pl.ds(0, CHUNK)], didx, isem).wait()

    def fetch_rows(k):
        sidx, _, _, rows, _, _, gsem, _, _ = slots[k]
        pltpu.async_copy(hp_s.at[sidx], rows, gsem)

    def process(i, k):
        sidx, didx, sdid, rows, srows, pb, gsem, ssem, _ = slots[k]
        pltpu.make_async_copy(hp_s.at[sidx], rows, gsem).wait()

        @pl.when(i < P2_CHUNKS - 1)
        def _():
            wait_idx(k ^ 1)
            fetch_rows(k ^ 1)

        @pl.when(i >= 2)
        def _():
            drain_scatters(k)

        def grp(g, c2):
            si = sidx[pl.ds(g * 16, 16)]
            di = didx[pl.ds(g * 16, 16)]
            sdid[pl.ds(g * 16, 16)] = di
            e = (plsc.load_gather(ab_v, [si + si])
                 + plsc.load_gather(ab_v, [di + di + 1]))
            e = jnp.where(e >= 0.0, e, 0.2 * e)
            pb[pl.ds(g * 16, 16)] = jnp.exp(e - cvec)
            return c2

        lax.fori_loop(0, GRP, grp, 0)

        def scale(g, c2):
            p16 = pb[pl.ds(g * 16, 16)]
            for j in range(16):
                pe = p16.at[jnp.full((16,), j, jnp.int32)].get(
                    mode="promise_in_bounds")
                e = g * 16 + j
                for cb in range(4):
                    srows[e, pl.ds(16 * cb, 16)] = (
                        rows[e, pl.ds(16 * cb, 16)] * pe)
            return c2

        lax.fori_loop(0, GRP, scale, 0)
        pltpu.async_copy(srows, embed_s.at[sdid], ssem, add=True)
        pltpu.async_copy(pb, denom_s.at[sdid], ssem, add=True)

        @pl.when(i < P2_CHUNKS - 2)
        def _():
            fetch_idx(i + 2, k)

    fetch_idx(0, 0)
    fetch_idx(1, 1)
    wait_idx(0)
    fetch_rows(0)

    def ring_iter(o, carry):
        process(o * 2, 0)
        process(o * 2 + 1, 1)
        return carry

    lax.fori_loop(0, P2_CHUNKS // 2, ring_iter, 0)
    drain_scatters(0)
    drain_scatters(1)

    plsc.subcore_barrier()
    pltpu.sync_copy(embed_s.at[pl.ds(sid * SLICE, SLICE)],
                    out_e.at[cid, pl.ds(sid * SLICE, SLICE)])
    pltpu.sync_copy(denom_s.at[pl.ds(sid * SLICE, SLICE)],
                    out_d.at[cid, pl.ds(sid * SLICE, SLICE)])


def _emb_body(nr, dn, bg, emb_ref):
    n = nr[0] + nr[1]
    d = dn[0:1, :] + dn[1:2, :] + 1e-16
    emb_ref[...] = n / jnp.transpose(d) + bg[...]


def _tc2_body(er_ref, ec_ref, out_ref):
    logits = lax.dot_general(er_ref[...], ec_ref[...], (((1,), (1,)), ((), ())),
                             preferred_element_type=jnp.float32)
    out_ref[...] = jax.nn.sigmoid(logits)


def kernel(x, adj, W_dense, b_dense, W_gat, att_src, att_dst, b_gat):
    f32 = jnp.float32
    loop = jnp.arange(N, dtype=adj.dtype)
    pad = E_PAD - E_TOT
    src = jnp.concatenate([adj[0].astype(jnp.int32), loop.astype(jnp.int32),
                           jnp.zeros((pad,), jnp.int32)])
    dst = jnp.concatenate([adj[1].astype(jnp.int32), loop.astype(jnp.int32),
                           jnp.full((pad,), N, jnp.int32)])

    hp, ab, c_c = pl.pallas_call(
        _tc1_body,
        out_shape=[
            jax.ShapeDtypeStruct((N, EMB), f32),
            jax.ShapeDtypeStruct((N, 2), f32),
            jax.ShapeDtypeStruct((1, 16), f32),
        ],
    )(x, W_dense, b_dense.reshape(1, EMB), W_gat,
      jnp.stack([att_src, att_dst], axis=0))
    ab_flat = jnp.concatenate([ab, jnp.zeros((16, 2), f32)]).reshape(-1)

    mesh = plsc.VectorSubcoreMesh(core_axis_name="c", subcore_axis_name="s",
                                  num_cores=2, num_subcores=16)
    sc_fn = pl.kernel(
        _sc_body,
        out_type=[
            jax.ShapeDtypeStruct((2, NPAD, EMB), f32),
            jax.ShapeDtypeStruct((2, NPAD), f32),
        ],
        mesh=mesh,
        compiler_params=pltpu.CompilerParams(needs_layout_passes=False,
                                             use_tc_tiling_on_sc=False),
        scratch_types=[
            pltpu.VMEM((AB,), f32),
            pltpu.VMEM((CHUNK, EMB), f32),
            pltpu.VMEM((CHUNK, EMB), f32),
            pltpu.VMEM((CHUNK, EMB), f32),
            pltpu.VMEM((CHUNK, EMB), f32),
            pltpu.VMEM((CHUNK,), jnp.int32),
            pltpu.VMEM((CHUNK,), jnp.int32),
            pltpu.VMEM((CHUNK,), jnp.int32),
            pltpu.VMEM((CHUNK,), jnp.int32),
            pltpu.VMEM((CHUNK,), jnp.int32),
            pltpu.VMEM((CHUNK,), jnp.int32),
            pltpu.VMEM((CHUNK,), f32),
            pltpu.VMEM((CHUNK,), f32),
            pltpu.VMEM((16,), f32),
            pltpu.VMEM_SHARED((NPAD,), f32),
            pltpu.VMEM_SHARED((NPAD, EMB), f32),
            pltpu.VMEM_SHARED((N, EMB), f32),
            pltpu.SemaphoreType.DMA,
            pltpu.SemaphoreType.DMA,
            pltpu.SemaphoreType.DMA,
            pltpu.SemaphoreType.DMA,
            pltpu.SemaphoreType.DMA,
            pltpu.SemaphoreType.DMA,
        ],
    )
    z1 = jnp.zeros((SLICE,), f32)
    z2 = jnp.zeros((SLICE, EMB), f32)
    nums, dens = sc_fn(src, dst, ab_flat, hp, c_c, z1, z2)

    emb_full = pl.pallas_call(
        _emb_body,
        out_shape=jax.ShapeDtypeStruct((NPAD, EMB), f32),
    )(nums, dens, b_gat.reshape(1, EMB))

    out = pl.pallas_call(
        _tc2_body,
        grid=(NPAD // BR, NPAD // BR),
        in_specs=[
            pl.BlockSpec((BR, EMB), lambda i, j: (i, 0)),
            pl.BlockSpec((BR, EMB), lambda i, j: (j, 0)),
        ],
        out_specs=pl.BlockSpec((BR, BR), lambda i, j: (i, j)),
        out_shape=jax.ShapeDtypeStruct((N, N), f32),
    )(emb_full, emb_full)
    return (out, emb_full[:N])
